# baseline (speedup 1.0000x reference)
import functools

import numpy as np
import jax
import jax.numpy as jnp
from jax import lax
from jax.experimental import pallas as pl
from jax.experimental.pallas import tpu as pltpu
from jax.experimental.pallas import tpu_sc as plsc

N_ATOMS = 10000
N_EDGES = 320000
N_MOL = 256
HIDDEN = 128
NUM_INTERACTIONS = 6
NUM_GAUSSIANS = 50
CUTOFF = 5.0
MEAN = 0.0
STD = 1.0
MAX_Z = 100

_LN2 = float(np.log(2.0))

_NC = 2
_NS = 16
_NW = _NC * _NS

_E_CHUNK = 80


def _ssp(x):
    return jnp.maximum(x, 0.0) + jnp.log1p(jnp.exp(-jnp.abs(x))) - _LN2


_D2_CHUNK = 400


def _sc_edge_d2(px, py, pz, jidx, iidx):
    per_w = N_EDGES // _NW
    n_chunks = per_w // _D2_CHUNK
    mesh = plsc.VectorSubcoreMesh(
        core_axis_name="c", subcore_axis_name="s",
        num_cores=_NC, num_subcores=_NS)

    @functools.partial(
        pl.kernel,
        out_type=jax.ShapeDtypeStruct((N_EDGES,), jnp.float32),
        mesh=mesh,
        scratch_types=[
            pltpu.VMEM((N_ATOMS,), jnp.float32),
            pltpu.VMEM((N_ATOMS,), jnp.float32),
            pltpu.VMEM((N_ATOMS,), jnp.float32),
            pltpu.VMEM((_D2_CHUNK,), jnp.int32),
            pltpu.VMEM((_D2_CHUNK,), jnp.int32),
            pltpu.VMEM((_D2_CHUNK,), jnp.float32),
        ],
        compiler_params=pltpu.CompilerParams(needs_layout_passes=False),
    )
    def k(px_hbm, py_hbm, pz_hbm, j_hbm, i_hbm, out_hbm,
          pxv, pyv, pzv, jv, iv, ob):
        c = lax.axis_index("c")
        s = lax.axis_index("s")
        pltpu.sync_copy(px_hbm, pxv)
        pltpu.sync_copy(py_hbm, pyv)
        pltpu.sync_copy(pz_hbm, pzv)
        base = (c * _NS + s) * per_w

        def body(kk, carry):
            off = base + kk * _D2_CHUNK
            pltpu.sync_copy(j_hbm.at[pl.ds(off, _D2_CHUNK)], jv)
            pltpu.sync_copy(i_hbm.at[pl.ds(off, _D2_CHUNK)], iv)

            def inner(q, carry2):
                sl = pl.ds(q * 16, 16)
                j16 = jv[sl]
                i16 = iv[sl]
                dx = plsc.load_gather(pxv, [j16]) - plsc.load_gather(pxv, [i16])
                dy = plsc.load_gather(pyv, [j16]) - plsc.load_gather(pyv, [i16])
                dz = plsc.load_gather(pzv, [j16]) - plsc.load_gather(pzv, [i16])
                ob[sl] = dx * dx + dy * dy + dz * dz
                return carry2

            lax.fori_loop(0, _D2_CHUNK // 16, inner, 0)
            pltpu.sync_copy(ob, out_hbm.at[pl.ds(off, _D2_CHUNK)])
            return carry

        lax.fori_loop(0, n_chunks, body, 0)

    return k(px, py, pz, jidx, iidx)


_MSG_CHUNK = 40


def _sc_msg_pass(xl, w_edge, jidx, iidx, zeros_nodes):
    per_w = N_EDGES // _NW
    n_chunks = per_w // _MSG_CHUNK
    nx, nw, ni = 3, 2, 4
    mesh = plsc.VectorSubcoreMesh(
        core_axis_name="c", subcore_axis_name="s",
        num_cores=_NC, num_subcores=_NS)

    buf_t = pltpu.VMEM((_MSG_CHUNK, HIDDEN), jnp.float32)
    wlen = _MSG_CHUNK * HIDDEN // 2
    wbuf_t = pltpu.VMEM((wlen,), jnp.int32)
    idx_t = pltpu.VMEM((_MSG_CHUNK,), jnp.int32)

    @functools.partial(
        pl.kernel,
        out_type=jax.ShapeDtypeStruct((_NC, N_ATOMS, HIDDEN), jnp.float32),
        mesh=mesh,
        scratch_types=(
            [pltpu.VMEM_SHARED((N_ATOMS, HIDDEN), jnp.float32)]
            + [buf_t] * nx
            + [wbuf_t] * nw
            + [idx_t] * (2 * ni)
            + [pltpu.SemaphoreType.DMA] * (2 * nx + nw + ni)
        ),
        compiler_params=pltpu.CompilerParams(needs_layout_passes=False),
    )
    def k(xl_hbm, w_hbm, j_hbm, i_hbm, z_hbm, out_hbm, acc, *rest):
        p = 0
        xr = list(rest[p:p + nx]); p += nx
        wr = list(rest[p:p + nw]); p += nw
        jc = list(rest[p:p + ni]); p += ni
        ic = list(rest[p:p + ni]); p += ni
        sg = list(rest[p:p + nx]); p += nx
        sw = list(rest[p:p + nw]); p += nw
        ss = list(rest[p:p + nx]); p += nx
        si = list(rest[p:p + ni]); p += ni
        c = lax.axis_index("c")
        s = lax.axis_index("s")
        wid = c * _NS + s

        @pl.when(s == 0)
        def _():
            pltpu.sync_copy(z_hbm, acc)

        plsc.subcore_barrier()
        ebase = wid * per_w

        def issue_idx(cc, bi):
            off = ebase + cc * _MSG_CHUNK
            pltpu.async_copy(j_hbm.at[pl.ds(off, _MSG_CHUNK)], jc[bi], si[bi])
            pltpu.async_copy(i_hbm.at[pl.ds(off, _MSG_CHUNK)], ic[bi], si[bi])

        def wait_idx(bi):
            pltpu.make_async_copy(
                j_hbm.at[pl.ds(ebase, _MSG_CHUNK)], jc[bi], si[bi]).wait()
            pltpu.make_async_copy(
                i_hbm.at[pl.ds(ebase, _MSG_CHUNK)], ic[bi], si[bi]).wait()

        def issue_gw(cc, bx, bw, bi):
            pltpu.async_copy(xl_hbm.at[jc[bi]], xr[bx], sg[bx])
            pltpu.async_copy(
                w_hbm.at[pl.ds((ebase + cc * _MSG_CHUNK) * (HIDDEN // 2),
                               wlen)],
                wr[bw], sw[bw])

        def wait_gw(bx, bw):
            pltpu.make_async_copy(xl_hbm.at[jc[0]], xr[bx], sg[bx]).wait()
            pltpu.make_async_copy(
                w_hbm.at[pl.ds(0, wlen)], wr[bw], sw[bw]).wait()

        def wait_sc(bx):
            pltpu.make_async_copy(xr[bx], acc.at[ic[0]], ss[bx]).wait()

        def mul(bx, bw):
            himask = jnp.int32(-65536)

            def row(e, carry2):
                for g in range(HIDDEN // 32):
                    wi = wr[bw][pl.ds(e * (HIDDEN // 2) + g * 16, 16)]
                    lo = plsc.bitcast(wi << 16, jnp.float32)
                    hi = plsc.bitcast(wi & himask, jnp.float32)
                    sl_lo = pl.ds(g * 32, 16)
                    sl_hi = pl.ds(g * 32 + 16, 16)
                    xr[bx][e, sl_lo] = xr[bx][e, sl_lo] * lo
                    xr[bx][e, sl_hi] = xr[bx][e, sl_hi] * hi
                return carry2

            lax.fori_loop(0, _MSG_CHUNK, row, 0)

        def scat(cc, bx, bi):
            pltpu.async_copy(xr[bx], acc.at[ic[bi]], ss[bx], add=True)

        def step(cc, wait_prev, do_issue_idx, do_issue_gw):
            bx, bw, bi = cc % nx, cc % nw, cc % ni
            wait_gw(bx, bw)
            mul(bx, bw)
            if wait_prev:
                wait_sc((cc + 2) % nx)
            if do_issue_idx:
                issue_idx(cc + 3, (cc + 3) % ni)
            if do_issue_gw:
                wait_idx((cc + 2) % ni)
                issue_gw(cc + 2, (cc + 2) % nx, (cc + 2) % nw, (cc + 2) % ni)
            scat(cc, bx, bi)

        for cc in range(3):
            issue_idx(cc, cc)
        for cc in range(2):
            wait_idx(cc)
            issue_gw(cc, cc, cc, cc)

        uf = 12
        n_full = (n_chunks - 3) // uf

        def body(g, carry):
            for kq in range(uf):
                cc = g * uf + kq

                def _inner(cc=cc, kq=kq):
                    wait_gw(kq % nx, kq % nw)
                    mul(kq % nx, kq % nw)

                    @pl.when(cc >= 1)
                    def _():
                        wait_sc((kq + 2) % nx)

                    issue_idx(cc + 3, (kq + 3) % ni)
                    wait_idx((kq + 2) % ni)
                    issue_gw(cc + 2, (kq + 2) % nx, (kq + 2) % nw,
                             (kq + 2) % ni)
                    scat(cc, kq % nx, kq % ni)

                _inner()
            return carry

        lax.fori_loop(0, n_full, body, 0)
        for cc in range(n_full * uf, n_chunks):
            step(cc, True, cc + 3 < n_chunks, cc + 2 < n_chunks)
        wait_sc((n_chunks - 1) % nx)
        plsc.subcore_barrier()
        pltpu.sync_copy(acc.at[pl.ds(s * 624, 624), :],
                        out_hbm.at[c, pl.ds(s * 624, 624), :])

        @pl.when(s == _NS - 1)
        def _():
            pltpu.sync_copy(acc.at[pl.ds(9984, N_ATOMS - 9984), :],
                            out_hbm.at[c, pl.ds(9984, N_ATOMS - 9984), :])

    return k(xl, w_edge, jidx, iidx, zeros_nodes)


_EB = 4000
_NB = 2000


def _tc_edge_feats(d2col):
    nblk = N_EDGES // _EB

    def body(d2_ref, f_ref):
        d2 = d2_ref[...] + 1e-12
        dist = jnp.sqrt(d2)
        cenv = 0.5 * (jnp.cos(dist * (np.pi / CUTOFF)) + 1.0)
        one = jnp.ones_like(d2)
        zero = jnp.zeros((_EB, 4), jnp.float32)
        f_ref[...] = jnp.concatenate([d2, dist, one, cenv, zero], axis=1)

    return pl.pallas_call(
        body,
        grid=(nblk,),
        in_specs=[
            pl.BlockSpec((_EB, 1), lambda i: (i, 0)),
        ],
        out_specs=pl.BlockSpec((_EB, 8), lambda i: (i, 0)),
        out_shape=jax.ShapeDtypeStruct((N_EDGES, 8), jnp.float32),
    )(d2col)


def _tc_embed(zf, emb):
    nblk = N_ATOMS // _NB

    def body(z_ref, emb_ref, h_ref):
        ids = lax.broadcasted_iota(jnp.int32, (_NB, MAX_Z), 1).astype(jnp.float32)
        oh = (z_ref[...] == ids).astype(jnp.float32)
        h_ref[...] = jnp.dot(oh, emb_ref[...],
                             preferred_element_type=jnp.float32,
                             precision=lax.Precision.HIGHEST)

    return pl.pallas_call(
        body,
        grid=(nblk,),
        in_specs=[
            pl.BlockSpec((_NB, 1), lambda i: (i, 0)),
            pl.BlockSpec((MAX_Z, HIDDEN), lambda i: (0, 0)),
        ],
        out_specs=pl.BlockSpec((_NB, HIDDEN), lambda i: (i, 0)),
        out_shape=jax.ShapeDtypeStruct((N_ATOMS, HIDDEN), jnp.float32),
    )(zf, emb)


def _tc_filter(feats, gmat, w1, b1, w2, b2):
    nblk = N_EDGES // _EB

    step = CUTOFF / (NUM_GAUSSIANS - 1)
    coeff = -0.5 / step**2

    def body(f_ref, g_ref, w1_ref, b1_ref, w2_ref, b2_ref, o_ref):
        f = f_ref[...]
        dist = f[:, 1:2]
        off = lax.broadcasted_iota(
            jnp.int32, (_EB, NUM_GAUSSIANS), 1).astype(jnp.float32) * step
        delta = dist - off
        ea = jnp.exp(coeff * delta * delta)
        t1 = _ssp(jnp.dot(ea, w1_ref[...],
                          preferred_element_type=jnp.float32) + b1_ref[...])
        w = jnp.dot(t1, w2_ref[...],
                    preferred_element_type=jnp.float32) + b2_ref[...]
        o_ref[...] = (w * f[:, 3:4]).astype(jnp.bfloat16)

    return pl.pallas_call(
        body,
        grid=(nblk,),
        in_specs=[
            pl.BlockSpec((_EB, 8), lambda i: (i, 0)),
            pl.BlockSpec((8, NUM_GAUSSIANS), lambda i: (0, 0)),
            pl.BlockSpec((NUM_GAUSSIANS, HIDDEN), lambda i: (0, 0)),
            pl.BlockSpec((1, HIDDEN), lambda i: (0, 0)),
            pl.BlockSpec((HIDDEN, HIDDEN), lambda i: (0, 0)),
            pl.BlockSpec((1, HIDDEN), lambda i: (0, 0)),
        ],
        out_specs=pl.BlockSpec((_EB, HIDDEN), lambda i: (i, 0)),
        out_shape=jax.ShapeDtypeStruct((N_EDGES, HIDDEN), jnp.bfloat16),
    )(feats, gmat, w1, b1, w2, b2)


def _tc_matmul(x, w):
    nblk = N_ATOMS // _NB

    def body(x_ref, w_ref, o_ref):
        o_ref[...] = jnp.dot(x_ref[...], w_ref[...],
                             preferred_element_type=jnp.float32)

    return pl.pallas_call(
        body,
        grid=(nblk,),
        in_specs=[
            pl.BlockSpec((_NB, HIDDEN), lambda i: (i, 0)),
            pl.BlockSpec((HIDDEN, HIDDEN), lambda i: (0, 0)),
        ],
        out_specs=pl.BlockSpec((_NB, HIDDEN), lambda i: (i, 0)),
        out_shape=jax.ShapeDtypeStruct((N_ATOMS, HIDDEN), jnp.float32),
    )(x, w)


def _tc_update(parts, h, w2, b2, lw, lb):
    nblk = N_ATOMS // _NB

    def body(p0_ref, p1_ref, h_ref, w2_ref, b2_ref, lw_ref, lb_ref, o_ref):
        agg = p0_ref[0] + p1_ref[0]
        xc = _ssp(jnp.dot(agg, w2_ref[...],
                          preferred_element_type=jnp.float32) + b2_ref[...])
        xc = jnp.dot(xc, lw_ref[...],
                     preferred_element_type=jnp.float32) + lb_ref[...]
        o_ref[...] = h_ref[...] + xc

    return pl.pallas_call(
        body,
        grid=(nblk,),
        in_specs=[
            pl.BlockSpec((1, _NB, HIDDEN), lambda i: (0, i, 0)),
            pl.BlockSpec((1, _NB, HIDDEN), lambda i: (1, i, 0)),
            pl.BlockSpec((_NB, HIDDEN), lambda i: (i, 0)),
            pl.BlockSpec((HIDDEN, HIDDEN), lambda i: (0, 0)),
            pl.BlockSpec((1, HIDDEN), lambda i: (0, 0)),
            pl.BlockSpec((HIDDEN, HIDDEN), lambda i: (0, 0)),
            pl.BlockSpec((1, HIDDEN), lambda i: (0, 0)),
        ],
        out_specs=pl.BlockSpec((_NB, HIDDEN), lambda i: (i, 0)),
        out_shape=jax.ShapeDtypeStruct((N_ATOMS, HIDDEN), jnp.float32),
    )(parts, parts, h, w2, b2, lw, lb)


def _tc_readout(h, batchf, w1, b1, w2, b2):
    nblk = N_ATOMS // _NB
    hh = HIDDEN // 2

    def body(h_ref, bf_ref, w1_ref, b1_ref, w2_ref, b2_ref, o_ref):
        i = pl.program_id(0)
        hv = _ssp(jnp.dot(h_ref[...], w1_ref[...],
                          preferred_element_type=jnp.float32) + b1_ref[...])
        ev = jnp.dot(hv, w2_ref[...],
                     preferred_element_type=jnp.float32) + b2_ref[...]
        ids = lax.broadcasted_iota(jnp.int32, (_NB, N_MOL), 1).astype(jnp.float32)
        oh = (bf_ref[...] == ids).astype(jnp.float32)
        contrib = lax.dot_general(oh, ev, (((0,), (0,)), ((), ())),
                                  preferred_element_type=jnp.float32)

        @pl.when(i == 0)
        def _():
            o_ref[...] = jnp.zeros_like(o_ref)

        o_ref[...] += contrib

    return pl.pallas_call(
        body,
        grid=(nblk,),
        in_specs=[
            pl.BlockSpec((_NB, HIDDEN), lambda i: (i, 0)),
            pl.BlockSpec((_NB, 1), lambda i: (i, 0)),
            pl.BlockSpec((HIDDEN, hh), lambda i: (0, 0)),
            pl.BlockSpec((1, hh), lambda i: (0, 0)),
            pl.BlockSpec((hh, 1), lambda i: (0, 0)),
            pl.BlockSpec((1, 1), lambda i: (0, 0)),
        ],
        out_specs=pl.BlockSpec((N_MOL, 1), lambda i: (0, 0)),
        out_shape=jax.ShapeDtypeStruct((N_MOL, 1), jnp.float32),
    )(h, batchf, w1, b1, w2, b2)


def _halfword_perm():
    perm = np.empty(HIDDEN, np.int64)
    for g in range(HIDDEN // 32):
        for m in range(16):
            perm[32 * g + 2 * m] = 32 * g + m
            perm[32 * g + 2 * m + 1] = 32 * g + 16 + m
    return perm


def _gauss_expand_mat():
    off = np.linspace(0.0, CUTOFF, NUM_GAUSSIANS).astype(np.float64)
    coeff = -0.5 / (off[1] - off[0]) ** 2
    g = np.zeros((8, NUM_GAUSSIANS), np.float32)
    g[0, :] = coeff
    g[1, :] = -2.0 * coeff * off
    g[2, :] = coeff * off * off
    return jnp.asarray(g)


def kernel(z, pos, batch, edge_index, emb, mlp_w1, mlp_b1, mlp_w2, mlp_b2,
           conv_lin1_w, conv_lin2_w, conv_lin2_b, lin_w, lin_b,
           out_w1, out_b1, out_w2, out_b2):
    f32 = jnp.float32
    jidx = edge_index[0].astype(jnp.int32)
    iidx = edge_index[1].astype(jnp.int32)
    posf = pos.astype(f32)
    zeros_nodes = jnp.zeros((N_ATOMS, HIDDEN), f32)
    gmat = _gauss_expand_mat()

    d2 = _sc_edge_d2(posf[:, 0], posf[:, 1], posf[:, 2], jidx, iidx)
    feats = _tc_edge_feats(d2.reshape(N_EDGES, 1))
    h = _tc_embed(z.astype(f32).reshape(N_ATOMS, 1), emb)

    perm = _halfword_perm()

    def filt(t):
        wb = _tc_filter(feats, gmat,
                        mlp_w1[t], mlp_b1[t].reshape(1, HIDDEN),
                        mlp_w2[t][:, perm], mlp_b2[t][perm].reshape(1, HIDDEN))
        return jax.lax.bitcast_convert_type(
            wb.reshape(-1, 2), jnp.int32)

    w_edge = filt(0)
    for t in range(NUM_INTERACTIONS):
        xl = _tc_matmul(h, conv_lin1_w[t])
        parts = _sc_msg_pass(xl, w_edge, jidx, iidx, zeros_nodes)
        if t + 1 < NUM_INTERACTIONS:
            w_edge = filt(t + 1)
        h = _tc_update(parts, h,
                       conv_lin2_w[t], conv_lin2_b[t].reshape(1, HIDDEN),
                       lin_w[t], lin_b[t].reshape(1, HIDDEN))

    energy = _tc_readout(h, batch.astype(f32).reshape(N_ATOMS, 1),
                         out_w1, out_b1.reshape(1, HIDDEN // 2),
                         out_w2, out_b2.reshape(1, 1))
    return energy[:, 0] * STD + MEAN


# speedup vs baseline: 32.9663x; 32.9663x over previous
import functools

import numpy as np
import jax
import jax.numpy as jnp
from jax import lax
from jax.experimental import pallas as pl
from jax.experimental.pallas import tpu as pltpu
from jax.experimental.pallas import tpu_sc as plsc

N_ATOMS = 10000
N_EDGES = 320000
N_MOL = 256
HIDDEN = 128
NUM_INTERACTIONS = 6
NUM_GAUSSIANS = 50
CUTOFF = 5.0
MEAN = 0.0
STD = 1.0
MAX_Z = 100

_LN2 = float(np.log(2.0))

_NC = 2
_NS = 16
_NW = _NC * _NS

_E_CHUNK = 80


def _ssp(x):
    return jnp.maximum(x, 0.0) + jnp.log1p(jnp.exp(-jnp.abs(x))) - _LN2


_D2_CHUNK = 400


def _sc_edge_d2(px, py, pz, jidx, iidx):
    per_w = N_EDGES // _NW
    n_chunks = per_w // _D2_CHUNK
    mesh = plsc.VectorSubcoreMesh(
        core_axis_name="c", subcore_axis_name="s",
        num_cores=_NC, num_subcores=_NS)

    @functools.partial(
        pl.kernel,
        out_type=jax.ShapeDtypeStruct((N_EDGES,), jnp.float32),
        mesh=mesh,
        scratch_types=[
            pltpu.VMEM((N_ATOMS,), jnp.float32),
            pltpu.VMEM((N_ATOMS,), jnp.float32),
            pltpu.VMEM((N_ATOMS,), jnp.float32),
            pltpu.VMEM((_D2_CHUNK,), jnp.int32),
            pltpu.VMEM((_D2_CHUNK,), jnp.int32),
            pltpu.VMEM((_D2_CHUNK,), jnp.float32),
        ],
        compiler_params=pltpu.CompilerParams(needs_layout_passes=False),
    )
    def k(px_hbm, py_hbm, pz_hbm, j_hbm, i_hbm, out_hbm,
          pxv, pyv, pzv, jv, iv, ob):
        c = lax.axis_index("c")
        s = lax.axis_index("s")
        pltpu.sync_copy(px_hbm, pxv)
        pltpu.sync_copy(py_hbm, pyv)
        pltpu.sync_copy(pz_hbm, pzv)
        base = (c * _NS + s) * per_w

        def body(kk, carry):
            off = base + kk * _D2_CHUNK
            pltpu.sync_copy(j_hbm.at[pl.ds(off, _D2_CHUNK)], jv)
            pltpu.sync_copy(i_hbm.at[pl.ds(off, _D2_CHUNK)], iv)

            def inner(q, carry2):
                sl = pl.ds(q * 16, 16)
                j16 = jv[sl]
                i16 = iv[sl]
                dx = plsc.load_gather(pxv, [j16]) - plsc.load_gather(pxv, [i16])
                dy = plsc.load_gather(pyv, [j16]) - plsc.load_gather(pyv, [i16])
                dz = plsc.load_gather(pzv, [j16]) - plsc.load_gather(pzv, [i16])
                ob[sl] = dx * dx + dy * dy + dz * dz
                return carry2

            lax.fori_loop(0, _D2_CHUNK // 16, inner, 0)
            pltpu.sync_copy(ob, out_hbm.at[pl.ds(off, _D2_CHUNK)])
            return carry

        lax.fori_loop(0, n_chunks, body, 0)

    return k(px, py, pz, jidx, iidx)


_MSG_CHUNK = 40


def _sc_msg_pass(xl, w_edge, jidx, iidx, zeros_nodes):
    per_w = N_EDGES // _NW
    n_chunks = per_w // _MSG_CHUNK
    nx, nw, ni = 3, 2, 4
    mesh = plsc.VectorSubcoreMesh(
        core_axis_name="c", subcore_axis_name="s",
        num_cores=_NC, num_subcores=_NS)

    buf_t = pltpu.VMEM((_MSG_CHUNK, HIDDEN), jnp.float32)
    wbuf_t = pltpu.VMEM((_MSG_CHUNK, HIDDEN // 2), jnp.int32)
    idx_t = pltpu.VMEM((_MSG_CHUNK,), jnp.int32)

    @functools.partial(
        pl.kernel,
        out_type=jax.ShapeDtypeStruct((_NC, N_ATOMS, HIDDEN), jnp.float32),
        mesh=mesh,
        scratch_types=(
            [pltpu.VMEM_SHARED((N_ATOMS, HIDDEN), jnp.float32)]
            + [buf_t] * nx
            + [wbuf_t] * nw
            + [idx_t] * (2 * ni)
            + [pltpu.SemaphoreType.DMA] * (2 * nx + nw + ni)
        ),
        compiler_params=pltpu.CompilerParams(needs_layout_passes=False),
    )
    def k(xl_hbm, w_hbm, j_hbm, i_hbm, z_hbm, out_hbm, acc, *rest):
        p = 0
        xr = list(rest[p:p + nx]); p += nx
        wr = list(rest[p:p + nw]); p += nw
        jc = list(rest[p:p + ni]); p += ni
        ic = list(rest[p:p + ni]); p += ni
        sg = list(rest[p:p + nx]); p += nx
        sw = list(rest[p:p + nw]); p += nw
        ss = list(rest[p:p + nx]); p += nx
        si = list(rest[p:p + ni]); p += ni
        c = lax.axis_index("c")
        s = lax.axis_index("s")
        wid = c * _NS + s

        @pl.when(s == 0)
        def _():
            pltpu.sync_copy(z_hbm, acc)

        plsc.subcore_barrier()
        ebase = wid * per_w

        def issue_idx(cc, bi):
            off = ebase + cc * _MSG_CHUNK
            pltpu.async_copy(j_hbm.at[pl.ds(off, _MSG_CHUNK)], jc[bi], si[bi])
            pltpu.async_copy(i_hbm.at[pl.ds(off, _MSG_CHUNK)], ic[bi], si[bi])

        def wait_idx(bi):
            pltpu.make_async_copy(
                j_hbm.at[pl.ds(ebase, _MSG_CHUNK)], jc[bi], si[bi]).wait()
            pltpu.make_async_copy(
                i_hbm.at[pl.ds(ebase, _MSG_CHUNK)], ic[bi], si[bi]).wait()

        def issue_gw(cc, bx, bw, bi):
            pltpu.async_copy(xl_hbm.at[jc[bi]], xr[bx], sg[bx])
            pltpu.async_copy(
                w_hbm.at[pl.ds(ebase + cc * _MSG_CHUNK, _MSG_CHUNK), :],
                wr[bw], sw[bw])

        def wait_gw(bx, bw):
            pltpu.make_async_copy(xl_hbm.at[jc[0]], xr[bx], sg[bx]).wait()
            pltpu.make_async_copy(
                w_hbm.at[pl.ds(ebase, _MSG_CHUNK), :], wr[bw], sw[bw]).wait()

        def wait_sc(bx):
            pltpu.make_async_copy(xr[bx], acc.at[ic[0]], ss[bx]).wait()

        def mul(bx, bw):
            himask = jnp.int32(-65536)

            def row(e, carry2):
                for g in range(HIDDEN // 32):
                    wi = wr[bw][e, pl.ds(g * 16, 16)]
                    lo = plsc.bitcast(wi << 16, jnp.float32)
                    hi = plsc.bitcast(wi & himask, jnp.float32)
                    sl_lo = pl.ds(g * 32, 16)
                    sl_hi = pl.ds(g * 32 + 16, 16)
                    xr[bx][e, sl_lo] = xr[bx][e, sl_lo] * lo
                    xr[bx][e, sl_hi] = xr[bx][e, sl_hi] * hi
                return carry2

            lax.fori_loop(0, _MSG_CHUNK, row, 0)

        def scat(cc, bx, bi):
            pltpu.async_copy(xr[bx], acc.at[ic[bi]], ss[bx], add=True)

        def step(cc, wait_prev, do_issue_idx, do_issue_gw):
            bx, bw, bi = cc % nx, cc % nw, cc % ni
            wait_gw(bx, bw)
            mul(bx, bw)
            if wait_prev:
                wait_sc((cc + 2) % nx)
            if do_issue_idx:
                issue_idx(cc + 3, (cc + 3) % ni)
            if do_issue_gw:
                wait_idx((cc + 2) % ni)
                issue_gw(cc + 2, (cc + 2) % nx, (cc + 2) % nw, (cc + 2) % ni)
            scat(cc, bx, bi)

        for cc in range(3):
            issue_idx(cc, cc)
        for cc in range(2):
            wait_idx(cc)
            issue_gw(cc, cc, cc, cc)

        uf = 12
        n_full = (n_chunks - 3) // uf

        def body(g, carry):
            for kq in range(uf):
                cc = g * uf + kq

                def _inner(cc=cc, kq=kq):
                    wait_gw(kq % nx, kq % nw)
                    mul(kq % nx, kq % nw)

                    @pl.when(cc >= 1)
                    def _():
                        wait_sc((kq + 2) % nx)

                    issue_idx(cc + 3, (kq + 3) % ni)
                    wait_idx((kq + 2) % ni)
                    issue_gw(cc + 2, (kq + 2) % nx, (kq + 2) % nw,
                             (kq + 2) % ni)
                    scat(cc, kq % nx, kq % ni)

                _inner()
            return carry

        lax.fori_loop(0, n_full, body, 0)
        for cc in range(n_full * uf, n_chunks):
            step(cc, True, cc + 3 < n_chunks, cc + 2 < n_chunks)
        wait_sc((n_chunks - 1) % nx)
        plsc.subcore_barrier()
        pltpu.sync_copy(acc.at[pl.ds(s * 624, 624), :],
                        out_hbm.at[c, pl.ds(s * 624, 624), :])

        @pl.when(s == _NS - 1)
        def _():
            pltpu.sync_copy(acc.at[pl.ds(9984, N_ATOMS - 9984), :],
                            out_hbm.at[c, pl.ds(9984, N_ATOMS - 9984), :])

    return k(xl, w_edge, jidx, iidx, zeros_nodes)


_EB = 4000
_NB = 2000


def _tc_edge_feats(d2col):
    nblk = N_EDGES // _EB

    def body(d2_ref, f_ref):
        d2 = d2_ref[...] + 1e-12
        dist = jnp.sqrt(d2)
        cenv = 0.5 * (jnp.cos(dist * (np.pi / CUTOFF)) + 1.0)
        one = jnp.ones_like(d2)
        zero = jnp.zeros((_EB, 4), jnp.float32)
        f_ref[...] = jnp.concatenate([d2, dist, one, cenv, zero], axis=1)

    return pl.pallas_call(
        body,
        grid=(nblk,),
        in_specs=[
            pl.BlockSpec((_EB, 1), lambda i: (i, 0)),
        ],
        out_specs=pl.BlockSpec((_EB, 8), lambda i: (i, 0)),
        out_shape=jax.ShapeDtypeStruct((N_EDGES, 8), jnp.float32),
    )(d2col)


def _tc_embed(zf, emb):
    nblk = N_ATOMS // _NB

    def body(z_ref, emb_ref, h_ref):
        ids = lax.broadcasted_iota(jnp.int32, (_NB, MAX_Z), 1).astype(jnp.float32)
        oh = (z_ref[...] == ids).astype(jnp.float32)
        h_ref[...] = jnp.dot(oh, emb_ref[...],
                             preferred_element_type=jnp.float32,
                             precision=lax.Precision.HIGHEST)

    return pl.pallas_call(
        body,
        grid=(nblk,),
        in_specs=[
            pl.BlockSpec((_NB, 1), lambda i: (i, 0)),
            pl.BlockSpec((MAX_Z, HIDDEN), lambda i: (0, 0)),
        ],
        out_specs=pl.BlockSpec((_NB, HIDDEN), lambda i: (i, 0)),
        out_shape=jax.ShapeDtypeStruct((N_ATOMS, HIDDEN), jnp.float32),
    )(zf, emb)


def _tc_filter(feats, gmat, w1, b1, w2, b2):
    nblk = N_EDGES // _EB

    step = CUTOFF / (NUM_GAUSSIANS - 1)
    coeff = -0.5 / step**2

    def body(f_ref, g_ref, w1_ref, b1_ref, w2_ref, b2_ref, o_ref):
        f = f_ref[...]
        dist = f[:, 1:2]
        off = lax.broadcasted_iota(
            jnp.int32, (_EB, NUM_GAUSSIANS), 1).astype(jnp.float32) * step
        delta = dist - off
        ea = jnp.exp(coeff * delta * delta)
        t1 = _ssp(jnp.dot(ea, w1_ref[...],
                          preferred_element_type=jnp.float32) + b1_ref[...])
        w = jnp.dot(t1, w2_ref[...],
                    preferred_element_type=jnp.float32) + b2_ref[...]
        w = w * f[:, 3:4]
        u = lax.bitcast_convert_type(w, jnp.int32)
        r = (u + jnp.int32(0x7FFF) + ((u >> 16) & jnp.int32(1))) >> 16
        a = r[:, : HIDDEN // 2] & jnp.int32(0xFFFF)
        b = r[:, HIDDEN // 2:] << 16
        o_ref[...] = a | b

    return pl.pallas_call(
        body,
        grid=(nblk,),
        in_specs=[
            pl.BlockSpec((_EB, 8), lambda i: (i, 0)),
            pl.BlockSpec((8, NUM_GAUSSIANS), lambda i: (0, 0)),
            pl.BlockSpec((NUM_GAUSSIANS, HIDDEN), lambda i: (0, 0)),
            pl.BlockSpec((1, HIDDEN), lambda i: (0, 0)),
            pl.BlockSpec((HIDDEN, HIDDEN), lambda i: (0, 0)),
            pl.BlockSpec((1, HIDDEN), lambda i: (0, 0)),
        ],
        out_specs=pl.BlockSpec((_EB, HIDDEN // 2), lambda i: (i, 0)),
        out_shape=jax.ShapeDtypeStruct((N_EDGES, HIDDEN // 2), jnp.int32),
    )(feats, gmat, w1, b1, w2, b2)


def _tc_matmul(x, w):
    nblk = N_ATOMS // _NB

    def body(x_ref, w_ref, o_ref):
        o_ref[...] = jnp.dot(x_ref[...], w_ref[...],
                             preferred_element_type=jnp.float32)

    return pl.pallas_call(
        body,
        grid=(nblk,),
        in_specs=[
            pl.BlockSpec((_NB, HIDDEN), lambda i: (i, 0)),
            pl.BlockSpec((HIDDEN, HIDDEN), lambda i: (0, 0)),
        ],
        out_specs=pl.BlockSpec((_NB, HIDDEN), lambda i: (i, 0)),
        out_shape=jax.ShapeDtypeStruct((N_ATOMS, HIDDEN), jnp.float32),
    )(x, w)


def _tc_update(parts, h, w2, b2, lw, lb):
    nblk = N_ATOMS // _NB

    def body(p0_ref, p1_ref, h_ref, w2_ref, b2_ref, lw_ref, lb_ref, o_ref):
        agg = p0_ref[0] + p1_ref[0]
        xc = _ssp(jnp.dot(agg, w2_ref[...],
                          preferred_element_type=jnp.float32) + b2_ref[...])
        xc = jnp.dot(xc, lw_ref[...],
                     preferred_element_type=jnp.float32) + lb_ref[...]
        o_ref[...] = h_ref[...] + xc

    return pl.pallas_call(
        body,
        grid=(nblk,),
        in_specs=[
            pl.BlockSpec((1, _NB, HIDDEN), lambda i: (0, i, 0)),
            pl.BlockSpec((1, _NB, HIDDEN), lambda i: (1, i, 0)),
            pl.BlockSpec((_NB, HIDDEN), lambda i: (i, 0)),
            pl.BlockSpec((HIDDEN, HIDDEN), lambda i: (0, 0)),
            pl.BlockSpec((1, HIDDEN), lambda i: (0, 0)),
            pl.BlockSpec((HIDDEN, HIDDEN), lambda i: (0, 0)),
            pl.BlockSpec((1, HIDDEN), lambda i: (0, 0)),
        ],
        out_specs=pl.BlockSpec((_NB, HIDDEN), lambda i: (i, 0)),
        out_shape=jax.ShapeDtypeStruct((N_ATOMS, HIDDEN), jnp.float32),
    )(parts, parts, h, w2, b2, lw, lb)


def _tc_readout(h, batchf, w1, b1, w2, b2):
    nblk = N_ATOMS // _NB
    hh = HIDDEN // 2

    def body(h_ref, bf_ref, w1_ref, b1_ref, w2_ref, b2_ref, o_ref):
        i = pl.program_id(0)
        hv = _ssp(jnp.dot(h_ref[...], w1_ref[...],
                          preferred_element_type=jnp.float32) + b1_ref[...])
        ev = jnp.dot(hv, w2_ref[...],
                     preferred_element_type=jnp.float32) + b2_ref[...]
        ids = lax.broadcasted_iota(jnp.int32, (_NB, N_MOL), 1).astype(jnp.float32)
        oh = (bf_ref[...] == ids).astype(jnp.float32)
        contrib = lax.dot_general(oh, ev, (((0,), (0,)), ((), ())),
                                  preferred_element_type=jnp.float32)

        @pl.when(i == 0)
        def _():
            o_ref[...] = jnp.zeros_like(o_ref)

        o_ref[...] += contrib

    return pl.pallas_call(
        body,
        grid=(nblk,),
        in_specs=[
            pl.BlockSpec((_NB, HIDDEN), lambda i: (i, 0)),
            pl.BlockSpec((_NB, 1), lambda i: (i, 0)),
            pl.BlockSpec((HIDDEN, hh), lambda i: (0, 0)),
            pl.BlockSpec((1, hh), lambda i: (0, 0)),
            pl.BlockSpec((hh, 1), lambda i: (0, 0)),
            pl.BlockSpec((1, 1), lambda i: (0, 0)),
        ],
        out_specs=pl.BlockSpec((N_MOL, 1), lambda i: (0, 0)),
        out_shape=jax.ShapeDtypeStruct((N_MOL, 1), jnp.float32),
    )(h, batchf, w1, b1, w2, b2)


def _halfword_perm():
    perm = np.empty(HIDDEN, np.int64)
    for g in range(HIDDEN // 32):
        for k in range(16):
            perm[16 * g + k] = 32 * g + k
            perm[64 + 16 * g + k] = 32 * g + 16 + k
    return perm


def _gauss_expand_mat():
    off = np.linspace(0.0, CUTOFF, NUM_GAUSSIANS).astype(np.float64)
    coeff = -0.5 / (off[1] - off[0]) ** 2
    g = np.zeros((8, NUM_GAUSSIANS), np.float32)
    g[0, :] = coeff
    g[1, :] = -2.0 * coeff * off
    g[2, :] = coeff * off * off
    return jnp.asarray(g)


def kernel(z, pos, batch, edge_index, emb, mlp_w1, mlp_b1, mlp_w2, mlp_b2,
           conv_lin1_w, conv_lin2_w, conv_lin2_b, lin_w, lin_b,
           out_w1, out_b1, out_w2, out_b2):
    f32 = jnp.float32
    jidx = edge_index[0].astype(jnp.int32)
    iidx = edge_index[1].astype(jnp.int32)
    posf = pos.astype(f32)
    zeros_nodes = jnp.zeros((N_ATOMS, HIDDEN), f32)
    gmat = _gauss_expand_mat()

    d2 = _sc_edge_d2(posf[:, 0], posf[:, 1], posf[:, 2], jidx, iidx)
    feats = _tc_edge_feats(d2.reshape(N_EDGES, 1))
    h = _tc_embed(z.astype(f32).reshape(N_ATOMS, 1), emb)

    perm = _halfword_perm()

    def filt(t):
        return _tc_filter(feats, gmat,
                          mlp_w1[t], mlp_b1[t].reshape(1, HIDDEN),
                          mlp_w2[t][:, perm],
                          mlp_b2[t][perm].reshape(1, HIDDEN))

    w_edge = filt(0)
    for t in range(NUM_INTERACTIONS):
        xl = _tc_matmul(h, conv_lin1_w[t])
        parts = _sc_msg_pass(xl, w_edge, jidx, iidx, zeros_nodes)
        if t + 1 < NUM_INTERACTIONS:
            w_edge = filt(t + 1)
        h = _tc_update(parts, h,
                       conv_lin2_w[t], conv_lin2_b[t].reshape(1, HIDDEN),
                       lin_w[t], lin_b[t].reshape(1, HIDDEN))

    energy = _tc_readout(h, batch.astype(f32).reshape(N_ATOMS, 1),
                         out_w1, out_b1.reshape(1, HIDDEN // 2),
                         out_w2, out_b2.reshape(1, 1))
    return energy[:, 0] * STD + MEAN
